# Initial kernel scaffold; baseline (speedup 1.0000x reference)
import jax
import jax.numpy as jnp
from jax.experimental import pallas as pl


def kernel(x, Wg, bg, g_norm, W1, b1, W2, b2):
    raise NotImplementedError("write your pallas kernel here")


# speedup vs baseline: 4.7856x; 4.7856x over previous
import jax
import jax.numpy as jnp
from jax.experimental import pallas as pl
from jax.experimental.pallas import tpu as pltpu

RMS_EPS = 1.1920929e-07
TEMP = 0.5
BT = 1024
EC = 2


def _moe_body(x_ref, wg_ref, bg_ref, gn_ref, w1_ref, b1_ref, w2_ref, b2_ref,
              o_ref, g_scr):
    e = pl.program_id(1)
    n_e = pl.num_programs(1)

    @pl.when(e == 0)
    def _compute_gate():
        logits = jnp.dot(x_ref[...], wg_ref[...],
                         preferred_element_type=jnp.float32) + bg_ref[...]
        ms = jnp.mean(logits * logits, axis=-1, keepdims=True)
        logits = logits / jnp.sqrt(ms + RMS_EPS) * gn_ref[...]
        z = logits * (1.0 / TEMP)
        m = jnp.max(z, axis=-1, keepdims=True)
        p = 1.0 / jnp.sum(jnp.exp(z - m), axis=-1, keepdims=True)
        g_scr[...] = p / (p + 1e-6)

    hs = [jnp.dot(x_ref[...], w1_ref[j], preferred_element_type=jnp.float32)
          + b1_ref[j] for j in range(EC)]
    acc = None
    for j in range(EC):
        h = hs[j]
        h = h * 0.5 * (1.0 + jax.lax.erf(h * 0.7071067811865476))
        c = jnp.dot(h.astype(jnp.bfloat16), w2_ref[j],
                    preferred_element_type=jnp.float32) + b2_ref[j]
        acc = c if acc is None else acc + c

    @pl.when(e == 0)
    def _init():
        o_ref[...] = acc

    @pl.when(e != 0)
    def _acc():
        o_ref[...] += acc

    @pl.when(e == n_e - 1)
    def _finish():
        o_ref[...] *= g_scr[...]


def kernel(x, Wg, bg, g_norm, W1, b1, W2, b2):
    B, S, D = x.shape
    E, _, DH = W1.shape
    T = B * S
    xf = x.reshape(T, D)

    grid = (T // BT, E // EC)
    out = pl.pallas_call(
        _moe_body,
        grid=grid,
        in_specs=[
            pl.BlockSpec((BT, D), lambda t, e: (t, 0)),
            pl.BlockSpec((D, E), lambda t, e: (0, 0)),
            pl.BlockSpec((1, E), lambda t, e: (0, 0)),
            pl.BlockSpec((1, E), lambda t, e: (0, 0)),
            pl.BlockSpec((EC, D, DH), lambda t, e: (e, 0, 0)),
            pl.BlockSpec((EC, 1, DH), lambda t, e: (e, 0, 0)),
            pl.BlockSpec((EC, DH, D), lambda t, e: (e, 0, 0)),
            pl.BlockSpec((EC, 1, D), lambda t, e: (e, 0, 0)),
        ],
        out_specs=pl.BlockSpec((BT, D), lambda t, e: (t, 0)),
        out_shape=jax.ShapeDtypeStruct((T, D), jnp.float32),
        scratch_shapes=[pltpu.VMEM((BT, 1), jnp.float32)],
        compiler_params=pltpu.CompilerParams(
            dimension_semantics=("parallel", "arbitrary"),
        ),
    )(xf, Wg, bg.reshape(1, E), g_norm.reshape(1, E), W1,
      b1.reshape(E, 1, DH), W2, b2.reshape(E, 1, D))
    return out.reshape(B, S, D)
